# Initial kernel scaffold; baseline (speedup 1.0000x reference)
import jax
import jax.numpy as jnp
from jax.experimental import pallas as pl


def kernel(image, depth):
    raise NotImplementedError("write your pallas kernel here")


# speedup vs baseline: 152.1139x; 152.1139x over previous
import jax
import jax.numpy as jnp
from jax import lax
from jax.experimental import pallas as pl
from jax.experimental.pallas import tpu as pltpu
from jax.experimental.pallas import tpu_sc as plsc

B, C, H, W = 8, 3, 512, 512
MAX_DISP = 32.0
NC, NS = 2, 16
NW = NC * NS
W_PER_B = NW // B
ROWS_PER_W = H // W_PER_B
RBLK = 16
NBLK = ROWS_PER_W // RBLK
NCH = W // 16


def _body(image_hbm, depth_hbm, out_l_hbm, out_r_hbm,
          depth_v, img_v, cnt_v, acc_v):
    wid = lax.axis_index("s") * NC + lax.axis_index("c")
    b = wid // W_PER_B
    y_base = (wid % W_PER_B) * ROWS_PER_W

    xiota = lax.iota(jnp.int32, 16)
    ones = jnp.ones((16,), jnp.float32)
    zeros = jnp.zeros((16,), jnp.float32)
    dvecs = [jnp.full((16,), di, jnp.int32) for di in range(2)]
    cvecs = [jnp.full((16,), c, jnp.int32) for c in range(C)]

    def do_block(blk, carry):
        ys = y_base + blk * RBLK
        pltpu.sync_copy(depth_hbm.at[b, pl.ds(ys, RBLK), :], depth_v)
        pltpu.sync_copy(image_hbm.at[b, :, pl.ds(ys, RBLK), :], img_v)

        def zero_k(k, c2):
            r = k // NCH
            xo = (k % NCH) * 16
            for di in range(2):
                cnt_v[di, r, pl.ds(xo, 16)] = zeros
                for c in range(C):
                    acc_v[di, c, r, pl.ds(xo, 16)] = zeros
            return c2
        lax.fori_loop(0, RBLK * NCH, zero_k, 0)

        def scat_k(k, c2):
            r = k // NCH
            xo = (k % NCH) * 16
            d16 = depth_v[r, pl.ds(xo, 16)]
            disp = (d16 * MAX_DISP + 0.5).astype(jnp.int32)
            xb = xiota + xo
            rr = jnp.broadcast_to(r, (16,))
            vals = [img_v[c, r, pl.ds(xo, 16)] for c in range(C)]
            for di in range(2):
                xt = xb - disp if di == 0 else xb + disp
                msk = (xt >= 0) & (xt < W)
                xtc = jnp.clip(xt, 0, W - 1)
                plsc.addupdate_scatter(cnt_v, [dvecs[di], rr, xtc],
                                       ones, mask=msk)
                for c in range(C):
                    plsc.addupdate_scatter(acc_v, [dvecs[di], cvecs[c], rr, xtc],
                                           vals[c], mask=msk)
            return c2
        lax.fori_loop(0, RBLK * NCH, scat_k, 0)

        def fin_k(k, c2):
            r = k // NCH
            xo = (k % NCH) * 16
            for di in range(2):
                cnt = cnt_v[di, r, pl.ds(xo, 16)]
                inv = 1.0 / jnp.maximum(cnt, 1.0)
                for c in range(C):
                    a = acc_v[di, c, r, pl.ds(xo, 16)]
                    acc_v[di, c, r, pl.ds(xo, 16)] = jnp.clip(a * inv, 0.0, 1.0)
            return c2
        lax.fori_loop(0, RBLK * NCH, fin_k, 0)

        pltpu.sync_copy(acc_v.at[0], out_l_hbm.at[b, :, pl.ds(ys, RBLK), :])
        pltpu.sync_copy(acc_v.at[1], out_r_hbm.at[b, :, pl.ds(ys, RBLK), :])
        return carry

    lax.fori_loop(0, NBLK, do_block, 0)


def kernel(image, depth):
    mesh = plsc.VectorSubcoreMesh(core_axis_name="c", subcore_axis_name="s",
                                  num_cores=NC, num_subcores=NS)
    f = pl.kernel(
        _body,
        out_type=(jax.ShapeDtypeStruct((B, C, H, W), jnp.float32),
                  jax.ShapeDtypeStruct((B, C, H, W), jnp.float32)),
        mesh=mesh,
        scratch_types=[
            pltpu.VMEM((RBLK, W), jnp.float32),
            pltpu.VMEM((C, RBLK, W), jnp.float32),
            pltpu.VMEM((2, RBLK, W), jnp.float32),
            pltpu.VMEM((2, C, RBLK, W), jnp.float32),
        ],
        compiler_params=pltpu.CompilerParams(use_tc_tiling_on_sc=False,
                                             needs_layout_passes=False),
    )
    return f(image, depth)
